# baseline (speedup 1.0000x reference)
import functools

import numpy as np
import jax
import jax.numpy as jnp
from jax import lax
from jax.experimental import pallas as pl
from jax.experimental.pallas import tpu as pltpu
from jax.experimental.pallas import tpu_sc as plsc

DIM = 64
NLANE = 16
NV = DIM // NLANE
CHUNK = 512
GSUB = CHUNK // 128

_EPS = 1e-5
_MAGIC = np.int32(0x5F3759DF)


def _rsqrt(a):
    i = plsc.bitcast(a, jnp.int32)
    i = _MAGIC - jax.lax.shift_right_logical(i, 1)
    y = plsc.bitcast(i, jnp.float32)
    half_a = 0.5 * a
    y = y * (1.5 - half_a * y * y)
    y = y * (1.5 - half_a * y * y)
    return y


def _lane_allreduce(x, lane_iota):
    for sh in (1, 2, 4, 8):
        x = x + jnp.take(x, lane_iota ^ sh)
    return x


def _make_sc_kernel(n_rows, n_workers):
    rw = n_rows // n_workers
    rw128 = rw // 128
    n_chunks = rw // CHUNK
    mesh = plsc.VectorSubcoreMesh(core_axis_name="c", subcore_axis_name="s")

    @functools.partial(
        pl.kernel,
        out_type=jax.ShapeDtypeStruct((n_rows, DIM), jnp.float32),
        mesh=mesh,
        scratch_types=[
            pltpu.VMEM((GSUB, 128), jnp.int32),
            pltpu.VMEM((CHUNK, DIM), jnp.float32),
            pltpu.VMEM((DIM,), jnp.float32),
            pltpu.VMEM((DIM,), jnp.float32),
            pltpu.SemaphoreType.DMA,
        ],
        compiler_params=pltpu.CompilerParams(
            needs_layout_passes=False, use_tc_tiling_on_sc=False),
    )
    def sc_kernel(idx_hbm, table_hbm, gamma_hbm, beta_hbm, out_hbm,
                  idx_v, rows_v, gam_v, bet_v, sem):
        wid = lax.axis_index("s") * 2 + lax.axis_index("c")
        base128 = wid * rw128
        pltpu.sync_copy(gamma_hbm, gam_v)
        pltpu.sync_copy(beta_hbm, bet_v)
        gb = tuple(gam_v[pl.ds(k * NLANE, NLANE)] for k in range(NV)) + \
             tuple(bet_v[pl.ds(k * NLANE, NLANE)] for k in range(NV))

        def chunk_body(g, carry):
            off128 = base128 + g * GSUB
            pltpu.sync_copy(idx_hbm.at[pl.ds(off128, GSUB)], idx_v)
            handles = [
                pltpu.async_copy(table_hbm.at[idx_v.at[j]],
                                 rows_v.at[pl.ds(j * 128, 128)], sem)
                for j in range(GSUB)
            ]
            for h in handles:
                h.wait()

            lane_iota = lax.iota(jnp.int32, NLANE)

            def row_body(r, c2):
                v = [rows_v[r, pl.ds(k * NLANE, NLANE)] for k in range(NV)]
                s4 = (v[0] + v[1]) + (v[2] + v[3])
                q4 = (v[0] * v[0] + v[1] * v[1]) + (v[2] * v[2] + v[3] * v[3])
                s = _lane_allreduce(s4, lane_iota)
                q = _lane_allreduce(q4, lane_iota)
                mean = s * (1.0 / DIM)
                var = q * (1.0 / DIM) - mean * mean
                y = _rsqrt(var + _EPS)
                for k in range(NV):
                    rows_v[r, pl.ds(k * NLANE, NLANE)] = (
                        (v[k] - mean) * y * c2[k] + c2[NV + k])
                return c2

            lax.fori_loop(0, CHUNK, row_body, gb, unroll=2)
            pltpu.sync_copy(rows_v, out_hbm.at[pl.ds(off128 * 128, CHUNK)])
            return carry

        lax.fori_loop(0, n_chunks, chunk_body, 0)

    return sc_kernel


def kernel(input_ids, table, gamma, beta):
    b, l = input_ids.shape
    v, d = table.shape
    assert d == DIM
    n = b * l
    n_workers = 32
    assert n % (n_workers * CHUNK) == 0
    idx2d = input_ids.reshape(n // 128, 128).astype(jnp.int32)
    out = _make_sc_kernel(n, n_workers)(idx2d, table, gamma, beta)
    return out.reshape(b, l, d)


# speedup vs baseline: 1.2153x; 1.2153x over previous
import functools

import numpy as np
import jax
import jax.numpy as jnp
from jax import lax
from jax.experimental import pallas as pl
from jax.experimental.pallas import tpu as pltpu
from jax.experimental.pallas import tpu_sc as plsc

DIM = 64
NLANE = 16
NV = DIM // NLANE
NB = 2
L_SPLIT = (128, 72)

_EPS = 1e-5
_MAGIC = np.int32(0x5F3759DF)


def _rsqrt(a):
    i = plsc.bitcast(a, jnp.int32)
    i = _MAGIC - lax.shift_right_logical(i, 1)
    y = plsc.bitcast(i, jnp.float32)
    half_a = 0.5 * a
    y = y * (1.5 - half_a * y * y)
    y = y * (1.5 - half_a * y * y)
    return y


def _lane_allreduce(x, lane_iota):
    for sh in (1, 2, 4, 8):
        x = x + jnp.take(x, lane_iota ^ sh)
    return x


def _make_sc_kernel(batch, seq):
    n_workers = 32
    b_per_w = batch // n_workers
    n_chunks = b_per_w // NB
    buf_bytes = NB * seq * DIM * 4
    mesh = plsc.VectorSubcoreMesh(core_axis_name="c", subcore_axis_name="s")

    @functools.partial(
        pl.kernel,
        out_type=jax.ShapeDtypeStruct((batch, seq, DIM), jnp.float32),
        mesh=mesh,
        scratch_types=[
            pltpu.VMEM((b_per_w, seq), jnp.int32),
            pltpu.VMEM((NB, seq, DIM), jnp.float32),
            pltpu.VMEM((NB, seq, DIM), jnp.float32),
            pltpu.VMEM((DIM,), jnp.float32),
            pltpu.VMEM((DIM,), jnp.float32),
            pltpu.SemaphoreType.DMA,
            pltpu.SemaphoreType.DMA,
            pltpu.SemaphoreType.DMA,
            pltpu.SemaphoreType.DMA,
        ],
        compiler_params=pltpu.CompilerParams(
            needs_layout_passes=False, use_tc_tiling_on_sc=False),
    )
    def sc_kernel(ids_hbm, table_hbm, gamma_hbm, beta_hbm, out_hbm,
                  idx_all, buf0, buf1, gam_v, bet_v,
                  gsem0, gsem1, ssem0, ssem1):
        wid = lax.axis_index("s") * 2 + lax.axis_index("c")
        b0 = wid * b_per_w
        pltpu.sync_copy(ids_hbm.at[pl.ds(b0, b_per_w)], idx_all)
        pltpu.sync_copy(gamma_hbm, gam_v)
        pltpu.sync_copy(beta_hbm, bet_v)
        gb = tuple(gam_v[pl.ds(k * NLANE, NLANE)] for k in range(NV)) + \
             tuple(bet_v[pl.ds(k * NLANE, NLANE)] for k in range(NV))
        lane_iota = lax.iota(jnp.int32, NLANE)

        def fire_gathers(g, buf, gsem):
            for b in range(NB):
                row = g * NB + b
                off = 0
                for ln in L_SPLIT:
                    pltpu.async_copy(
                        table_hbm.at[idx_all.at[row, pl.ds(off, ln)]],
                        buf.at[b, pl.ds(off, ln)], gsem)
                    off += ln

        def drain_gathers(buf, gsem):
            pltpu.make_async_copy(out_hbm.at[pl.ds(0, NB)], buf, gsem).wait()

        def drain_stores(buf, ssem):
            pltpu.make_async_copy(buf, out_hbm.at[pl.ds(0, NB)], ssem).wait()

        def compute_and_store(g, buf, ssem):
            for b in range(NB):
                def row_body(l, c2):
                    v = [buf[b, l, pl.ds(k * NLANE, NLANE)] for k in range(NV)]
                    s4 = (v[0] + v[1]) + (v[2] + v[3])
                    q4 = (v[0] * v[0] + v[1] * v[1]) + \
                         (v[2] * v[2] + v[3] * v[3])
                    s = _lane_allreduce(s4, lane_iota)
                    q = _lane_allreduce(q4, lane_iota)
                    mean = s * (1.0 / DIM)
                    var = q * (1.0 / DIM) - mean * mean
                    y = _rsqrt(var + _EPS)
                    for k in range(NV):
                        buf[b, l, pl.ds(k * NLANE, NLANE)] = (
                            (v[k] - mean) * y * c2[k] + c2[NV + k])
                    return c2

                lax.fori_loop(0, seq, row_body, gb, unroll=4)
                pltpu.async_copy(buf.at[b],
                                 out_hbm.at[b0 + g * NB + b], ssem)

        fire_gathers(0, buf0, gsem0)

        def chunk_pair(t, carry):
            ga = 2 * t

            @pl.when(t > 0)
            def _():
                drain_stores(buf1, ssem1)

            fire_gathers(ga + 1, buf1, gsem1)
            drain_gathers(buf0, gsem0)
            compute_and_store(ga, buf0, ssem0)

            drain_stores(buf0, ssem0)

            @pl.when(t < n_chunks // 2 - 1)
            def _():
                fire_gathers(ga + 2, buf0, gsem0)

            drain_gathers(buf1, gsem1)
            compute_and_store(ga + 1, buf1, ssem1)
            return carry

        lax.fori_loop(0, n_chunks // 2, chunk_pair, 0)
        drain_stores(buf1, ssem1)

    return sc_kernel


def kernel(input_ids, table, gamma, beta):
    b, l = input_ids.shape
    v, d = table.shape
    assert d == DIM and b % 32 == 0
    ids = input_ids.astype(jnp.int32)
    return _make_sc_kernel(b, l)(ids, table, gamma, beta)
